# Initial kernel scaffold; baseline (speedup 1.0000x reference)
import jax
import jax.numpy as jnp
from jax.experimental import pallas as pl


def kernel(hidden_states, token_types, cos, sin, Wq, bq, Wk, bk, Wv, bv, Wo):
    raise NotImplementedError("write your pallas kernel here")


# speedup vs baseline: 1.0402x; 1.0402x over previous
import functools

import jax
import jax.numpy as jnp
from jax.experimental import pallas as pl


def _rot(x):
    half = x.shape[-1] // 2
    return jnp.concatenate([-x[..., half:], x[..., :half]], axis=-1)


def _qkv_kernel(x_ref, tt_ref, Wq_ref, bq_ref, Wk_ref, bk_ref, Wv_ref,
                bv_ref, q_ref, k_ref, v_ref):
    x = x_ref[...]
    sel = tt_ref[...] == 1

    def proj(W_ref, b_ref):
        y0 = jnp.dot(x, W_ref[0], preferred_element_type=jnp.float32)
        y1 = jnp.dot(x, W_ref[1], preferred_element_type=jnp.float32)
        y = jnp.where(sel, y1 + b_ref[1:2, :], y0 + b_ref[0:1, :])
        return y

    q_ref[...] = proj(Wq_ref, bq_ref)
    k_ref[...] = proj(Wk_ref, bk_ref)
    v_ref[...] = proj(Wv_ref, bv_ref)


def _attn_kernel(q_ref, k_ref, v_ref, cq_ref, sq_ref, ck_ref, sk_ref,
                 o_ref, *, bq_blk, seq, scale):
    i = pl.program_id(1)
    q = q_ref[0]
    k = k_ref[0]
    v = v_ref[0]
    cq = cq_ref[...]
    sq = sq_ref[...]
    q = q * cq + _rot(q) * sq
    k = k * ck_ref[...] + _rot(k) * sk_ref[...]

    s = jnp.dot(q, k.T, preferred_element_type=jnp.float32) * scale
    row = i * bq_blk + jax.lax.broadcasted_iota(jnp.int32, (bq_blk, seq), 0)
    col = jax.lax.broadcasted_iota(jnp.int32, (bq_blk, seq), 1)
    s = jnp.where(col <= row, s, -jnp.inf)
    m = jnp.max(s, axis=-1, keepdims=True)
    p = jnp.exp(s - m)
    l = jnp.sum(p, axis=-1, keepdims=True)
    o_ref[0] = jnp.dot(p, v, preferred_element_type=jnp.float32) / l


def _oproj_kernel(x_ref, tt_ref, Wo_ref, o_ref):
    x = x_ref[...]
    sel = tt_ref[...] == 1
    y0 = jnp.dot(x, Wo_ref[0], preferred_element_type=jnp.float32)
    y1 = jnp.dot(x, Wo_ref[1], preferred_element_type=jnp.float32)
    o_ref[...] = jnp.where(sel, y1, y0)


def kernel(hidden_states, token_types, cos, sin, Wq, bq, Wk, bk, Wv, bv, Wo):
    bsz, seq, d = hidden_states.shape
    dh = cos.shape[-1]
    h = Wq.shape[2] // dh
    kv = Wk.shape[2] // dh
    nrep = h // kv
    scale = 1.0 / float(dh) ** 0.5

    x = hidden_states.reshape(seq, d)
    tt = token_types.reshape(seq, 1).astype(jnp.int32)
    cs = cos.reshape(seq, dh)
    sn = sin.reshape(seq, dh)

    BT = 256
    nt = seq // BT
    full3 = lambda shp: pl.BlockSpec(shp, lambda i: (0, 0, 0))
    full2 = lambda shp: pl.BlockSpec(shp, lambda i: (0, 0))

    q2d, k2d, v2d = pl.pallas_call(
        _qkv_kernel,
        grid=(nt,),
        in_specs=[
            pl.BlockSpec((BT, d), lambda i: (i, 0)),
            pl.BlockSpec((BT, 1), lambda i: (i, 0)),
            full3(Wq.shape), full2(bq.shape),
            full3(Wk.shape), full2(bk.shape),
            full3(Wv.shape), full2(bv.shape),
        ],
        out_specs=[
            pl.BlockSpec((BT, h * dh), lambda i: (i, 0)),
            pl.BlockSpec((BT, kv * dh), lambda i: (i, 0)),
            pl.BlockSpec((BT, kv * dh), lambda i: (i, 0)),
        ],
        out_shape=[
            jax.ShapeDtypeStruct((seq, h * dh), jnp.float32),
            jax.ShapeDtypeStruct((seq, kv * dh), jnp.float32),
            jax.ShapeDtypeStruct((seq, kv * dh), jnp.float32),
        ],
    )(x, tt, Wq, bq, Wk, bk, Wv, bv)

    q3 = q2d.reshape(seq, h, dh).transpose(1, 0, 2)
    k3 = k2d.reshape(seq, kv, dh).transpose(1, 0, 2)
    v3 = v2d.reshape(seq, kv, dh).transpose(1, 0, 2)

    BQ = 512
    nq = seq // BQ
    attn3 = pl.pallas_call(
        functools.partial(_attn_kernel, bq_blk=BQ, seq=seq, scale=scale),
        grid=(h, nq),
        in_specs=[
            pl.BlockSpec((1, BQ, dh), lambda hh, i: (hh, i, 0)),
            pl.BlockSpec((1, seq, dh), lambda hh, i: (hh // nrep, 0, 0)),
            pl.BlockSpec((1, seq, dh), lambda hh, i: (hh // nrep, 0, 0)),
            pl.BlockSpec((BQ, dh), lambda hh, i: (i, 0)),
            pl.BlockSpec((BQ, dh), lambda hh, i: (i, 0)),
            pl.BlockSpec((seq, dh), lambda hh, i: (0, 0)),
            pl.BlockSpec((seq, dh), lambda hh, i: (0, 0)),
        ],
        out_specs=pl.BlockSpec((1, BQ, dh), lambda hh, i: (hh, i, 0)),
        out_shape=jax.ShapeDtypeStruct((h, seq, dh), jnp.float32),
    )(q3, k3, v3, cs, sn, cs, sn)

    attn2d = attn3.transpose(1, 0, 2).reshape(seq, h * dh)

    out = pl.pallas_call(
        _oproj_kernel,
        grid=(nt,),
        in_specs=[
            pl.BlockSpec((BT, h * dh), lambda i: (i, 0)),
            pl.BlockSpec((BT, 1), lambda i: (i, 0)),
            full3(Wo.shape),
        ],
        out_specs=pl.BlockSpec((BT, d), lambda i: (i, 0)),
        out_shape=jax.ShapeDtypeStruct((seq, d), jnp.float32),
    )(attn2d, tt, Wo)

    return out.reshape(bsz, seq, d)


# speedup vs baseline: 1.9038x; 1.8302x over previous
import functools

import jax
import jax.numpy as jnp
from jax.experimental import pallas as pl
from jax.experimental.pallas import tpu as pltpu


def _rope(x, c, s):
    half = x.shape[-1] // 2
    rot = jnp.concatenate([-x[..., half:], x[..., :half]], axis=-1)
    return x * c + rot * s


def _fused_kernel(x_ref, tt_ref, cos_ref, sin_ref, Wq_ref, bq_ref, Wk_ref,
                  bk_ref, Wv_ref, bv_ref, Wo_ref, o_ref, ksc, vsc, asc, *,
                  bq_blk, seq, h, kv, dh, scale):
    i = pl.program_id(0)
    nrep = h // kv
    x = x_ref[...]
    sel = tt_ref[...] == 1
    c = cos_ref[...]
    s = sin_ref[...]

    def proj(W_ref, b_ref):
        y0 = jnp.dot(x, W_ref[0], preferred_element_type=jnp.float32)
        y1 = jnp.dot(x, W_ref[1], preferred_element_type=jnp.float32)
        return jnp.where(sel, y1 + b_ref[1:2, :], y0 + b_ref[0:1, :])

    @pl.when(i == 0)
    def _zero_scratch():
        vsc[...] = jnp.zeros_like(vsc)

    q = proj(Wq_ref, bq_ref)
    k = proj(Wk_ref, bk_ref)
    vsc[pl.ds(i * bq_blk, bq_blk), :] = proj(Wv_ref, bv_ref)
    for g in range(kv):
        ksc[pl.ds(i * bq_blk, bq_blk), g * dh:(g + 1) * dh] = (
            _rope(k[:, g * dh:(g + 1) * dh], c, s))

    row = i * bq_blk + jax.lax.broadcasted_iota(jnp.int32, (bq_blk, seq), 0)
    col = jax.lax.broadcasted_iota(jnp.int32, (bq_blk, seq), 1)
    mask = col <= row

    for hh in range(h):
        g = hh // nrep
        qh = _rope(q[:, hh * dh:(hh + 1) * dh], c, s)
        kg = ksc[:, g * dh:(g + 1) * dh]
        vg = vsc[:, g * dh:(g + 1) * dh]
        sc = jnp.dot(qh, kg.T, preferred_element_type=jnp.float32) * scale
        sc = jnp.where(mask, sc, -jnp.inf)
        m = jnp.max(sc, axis=-1, keepdims=True)
        p = jnp.exp(sc - m)
        l = jnp.sum(p, axis=-1, keepdims=True)
        asc[:, hh * dh:(hh + 1) * dh] = (
            jnp.dot(p, vg, preferred_element_type=jnp.float32) / l)

    attn = asc[...]
    y0 = jnp.dot(attn, Wo_ref[0], preferred_element_type=jnp.float32)
    y1 = jnp.dot(attn, Wo_ref[1], preferred_element_type=jnp.float32)
    o_ref[...] = jnp.where(sel, y1, y0)


def kernel(hidden_states, token_types, cos, sin, Wq, bq, Wk, bk, Wv, bv, Wo):
    bsz, seq, d = hidden_states.shape
    dh = cos.shape[-1]
    h = Wq.shape[2] // dh
    kv = Wk.shape[2] // dh
    scale = 1.0 / float(dh) ** 0.5

    x = hidden_states.reshape(seq, d)
    tt = token_types.reshape(seq, 1).astype(jnp.int32)
    cs = cos.reshape(seq, dh)
    sn = sin.reshape(seq, dh)

    BQ = 256
    nq = seq // BQ
    full3 = lambda shp: pl.BlockSpec(shp, lambda i: (0, 0, 0))
    full2 = lambda shp: pl.BlockSpec(shp, lambda i: (0, 0))

    out = pl.pallas_call(
        functools.partial(_fused_kernel, bq_blk=BQ, seq=seq, h=h, kv=kv,
                          dh=dh, scale=scale),
        grid=(nq,),
        in_specs=[
            pl.BlockSpec((BQ, d), lambda i: (i, 0)),
            pl.BlockSpec((BQ, 1), lambda i: (i, 0)),
            pl.BlockSpec((BQ, dh), lambda i: (i, 0)),
            pl.BlockSpec((BQ, dh), lambda i: (i, 0)),
            full3(Wq.shape), full2(bq.shape),
            full3(Wk.shape), full2(bk.shape),
            full3(Wv.shape), full2(bv.shape),
            full3(Wo.shape),
        ],
        out_specs=pl.BlockSpec((BQ, d), lambda i: (i, 0)),
        out_shape=jax.ShapeDtypeStruct((seq, d), jnp.float32),
        compiler_params=pltpu.CompilerParams(
            vmem_limit_bytes=63 * 1024 * 1024),
        scratch_shapes=[
            pltpu.VMEM((seq, kv * dh), jnp.float32),
            pltpu.VMEM((seq, kv * dh), jnp.float32),
            pltpu.VMEM((BQ, h * dh), jnp.float32),
        ],
    )(x, tt, cs, sn, Wq, bq, Wk, bk, Wv, bv, Wo)

    return out.reshape(bsz, seq, d)
